# Initial kernel scaffold; baseline (speedup 1.0000x reference)
import jax
import jax.numpy as jnp
from jax.experimental import pallas as pl


def kernel(x, table):
    raise NotImplementedError("write your pallas kernel here")


# speedup vs baseline: 4.2146x; 4.2146x over previous
import functools
import math

import jax
import jax.numpy as jnp
from jax import lax
from jax.experimental import pallas as pl
from jax.experimental.pallas import tpu as pltpu
from jax.experimental.pallas import tpu_sc as plsc

D_MODEL = 128
SCALE = math.sqrt(float(D_MODEL))
NUM_WORKERS = 32
CHUNK = 128
LANES = 16


def _make_kernel(n_rows: int):
    rows_per_worker = n_rows // NUM_WORKERS
    n_chunks = rows_per_worker // CHUNK
    mesh = plsc.VectorSubcoreMesh(core_axis_name="c", subcore_axis_name="s")

    @functools.partial(
        pl.kernel,
        out_type=jax.ShapeDtypeStruct((n_rows, D_MODEL), jnp.float32),
        mesh=mesh,
        scratch_types=[
            pltpu.VMEM((CHUNK,), jnp.int32),
            pltpu.VMEM((CHUNK, D_MODEL), jnp.float32),
            pltpu.SemaphoreType.DMA,
        ],
    )
    def gather_scale(x_hbm, table_hbm, out_hbm, idx_v, rows_v, sem):
        wid = lax.axis_index("s") * 2 + lax.axis_index("c")
        base = wid * rows_per_worker

        def chunk_body(ci, _):
            off = base + ci * CHUNK
            pltpu.sync_copy(x_hbm.at[pl.ds(off, CHUNK)], idx_v)
            pltpu.async_copy(table_hbm.at[idx_v], rows_v, sem).wait()

            def scale_row(i, _):
                for j in range(D_MODEL // LANES):
                    sl = pl.ds(j * LANES, LANES)
                    rows_v[i, sl] = rows_v[i, sl] * SCALE
                return 0

            lax.fori_loop(0, CHUNK, scale_row, 0)
            pltpu.sync_copy(rows_v, out_hbm.at[pl.ds(off, CHUNK)])
            return 0

        lax.fori_loop(0, n_chunks, chunk_body, 0)

    return gather_scale


def kernel(x, table):
    b, s = x.shape
    n_rows = b * s
    out = _make_kernel(n_rows)(x.reshape(n_rows).astype(jnp.int32), table)
    return out.reshape(b, s, D_MODEL)


# speedup vs baseline: 7.9919x; 1.8963x over previous
import functools
import math

import jax
import jax.numpy as jnp
from jax import lax
from jax.experimental import pallas as pl
from jax.experimental.pallas import tpu as pltpu
from jax.experimental.pallas import tpu_sc as plsc

D_MODEL = 128
SCALE = math.sqrt(float(D_MODEL))
NUM_WORKERS = 32
CHUNK = 128
LANES = 16
NBUF = 2


def _make_kernel(n_rows: int):
    rows_per_worker = n_rows // NUM_WORKERS
    n_chunks = rows_per_worker // CHUNK
    n_groups = n_chunks // NBUF
    assert n_chunks % NBUF == 0 and n_groups >= 3
    mesh = plsc.VectorSubcoreMesh(core_axis_name="c", subcore_axis_name="s")

    @functools.partial(
        pl.kernel,
        out_type=jax.ShapeDtypeStruct((n_rows, D_MODEL), jnp.float32),
        mesh=mesh,
        scratch_types=[
            pltpu.VMEM((n_chunks, CHUNK), jnp.int32),
            [pltpu.VMEM((CHUNK, D_MODEL), jnp.float32) for _ in range(NBUF)],
            [pltpu.VMEM((CHUNK, D_MODEL), jnp.float32) for _ in range(NBUF)],
            [pltpu.SemaphoreType.DMA for _ in range(NBUF)],
            [pltpu.SemaphoreType.DMA for _ in range(NBUF)],
        ],
    )
    def gather_scale(x_hbm, table_hbm, out_hbm, idx_v, bin, bout, gsem, ssem):
        wid = lax.axis_index("s") * 2 + lax.axis_index("c")
        base = wid * rows_per_worker
        pltpu.sync_copy(x_hbm.at[wid], idx_v)

        def start_gather(ci, b):
            pltpu.async_copy(table_hbm.at[idx_v.at[ci]], bin[b], gsem[b])

        def wait_gather(ci, b):
            pltpu.make_async_copy(table_hbm.at[idx_v.at[ci]], bin[b], gsem[b]).wait()

        def start_store(ci, b):
            pltpu.async_copy(bout[b], out_hbm.at[pl.ds(base + ci * CHUNK, CHUNK)],
                             ssem[b])

        def wait_store(b):
            pltpu.make_async_copy(bout[b], out_hbm.at[pl.ds(base, CHUNK)],
                                  ssem[b]).wait()

        def scale(b):
            @functools.partial(plsc.parallel_loop, 0, CHUNK, unroll=4)
            def _(i):
                for j in range(D_MODEL // LANES):
                    sl = pl.ds(j * LANES, LANES)
                    bout[b][i, sl] = bin[b][i, sl] * SCALE

        for b in range(NBUF):
            start_gather(b, b)

        for b in range(NBUF):
            wait_gather(b, b)
            scale(b)
            start_gather(NBUF + b, b)
            start_store(b, b)

        def group(g, _):
            ci0 = g * NBUF
            for b in range(NBUF):
                ci = ci0 + b
                wait_gather(ci, b)
                wait_store(b)
                scale(b)
                start_gather(ci + NBUF, b)
                start_store(ci, b)
            return 0

        lax.fori_loop(1, n_groups - 1, group, 0)

        ci0 = (n_groups - 1) * NBUF
        for b in range(NBUF):
            ci = ci0 + b
            wait_gather(ci, b)
            wait_store(b)
            scale(b)
            start_store(ci, b)
        for b in range(NBUF):
            wait_store(b)

    return gather_scale


def kernel(x, table):
    b, s = x.shape
    n_rows = b * s
    x_tiled = x.reshape(NUM_WORKERS, n_rows // (NUM_WORKERS * CHUNK), CHUNK)
    out = _make_kernel(n_rows)(x_tiled.astype(jnp.int32), table)
    return out.reshape(b, s, D_MODEL)
